# baseline (speedup 1.0000x reference)
import functools
import math

import jax
import jax.numpy as jnp
from jax import lax
from jax.experimental import pallas as pl
from jax.experimental.pallas import tpu as pltpu

_HALF_LOG_2PI = 0.5 * math.log(2.0 * math.pi)
_LOG2E = 1.4426950408889634
_LN2 = 0.6931471805599453
_LG = 8
_KC = 64


def _body(zt_ref, mu_ref, lv_ref, w_ref, out_ref, p3_s):
    K, L = mu_ref.shape
    Bb = zt_ref.shape[1]
    NG = L // _LG
    mu = mu_ref[...]
    lv = lv_ref[...]
    wv = w_ref[...]
    wmax = jnp.max(wv)
    logw = wv - wmax - jnp.log(jnp.sum(jnp.exp(wv - wmax)))
    prec = jnp.exp(-lv)
    a_all = _LOG2E * ((-_HALF_LOG_2PI) - 0.5 * lv
                      - 0.5 * prec * mu * mu + logw)
    b_all = _LOG2E * prec * mu
    c_all = (-0.5 * _LOG2E) * prec
    for g in range(NG):
        sl = slice(g * _LG, (g + 1) * _LG)
        p3_s[pl.ds(g, 1)] = jnp.concatenate(
            [a_all[:, sl], b_all[:, sl], c_all[:, sl]], axis=0)[None]

    def lgroup(g, _):
        pc = p3_s[pl.ds(g, 1)][0]
        for j in range(_LG):
            col = pc[:, j:j + 1]
            a = col[0:K]
            b = col[K:2 * K]
            c = col[2 * K:3 * K]
            zrow = zt_ref[pl.ds(g * _LG + j, 1), :]
            z2 = zrow * zrow
            m_run = None
            s_run = None
            for kc in range(K // _KC):
                ks = slice(kc * _KC, (kc + 1) * _KC)
                t2c = a[ks] + b[ks] * zrow + c[ks] * z2
                t3 = t2c.reshape(_KC // 8, 8, Bb)
                mc = jnp.max(t3, axis=0)
                sc = jnp.sum(jnp.exp2(t3 - mc[None]), axis=0)
                if m_run is None:
                    m_run, s_run = mc, sc
                else:
                    m_new = jnp.maximum(m_run, mc)
                    s_run = (s_run * jnp.exp2(m_run - m_new)
                             + sc * jnp.exp2(mc - m_new))
                    m_run = m_new
            m1 = jnp.max(m_run, axis=0, keepdims=True)
            s1 = jnp.sum(s_run * jnp.exp2(m_run - m1),
                         axis=0, keepdims=True)
            out_ref[pl.ds(g * _LG + j, 1), :] = _LN2 * (m1 + jnp.log2(s1))
        return 0

    lax.fori_loop(0, NG, lgroup, 0)


@jax.jit
def kernel(z, means, logvars, w):
    B, L = z.shape
    K = means.shape[0]
    zt = z.T
    w2 = w.reshape(K, 1)
    Bb = 256
    grid = (B // Bb,)
    out = pl.pallas_call(
        _body,
        grid=grid,
        in_specs=[
            pl.BlockSpec((L, Bb), lambda i: (0, i)),
            pl.BlockSpec((K, L), lambda i: (0, 0)),
            pl.BlockSpec((K, L), lambda i: (0, 0)),
            pl.BlockSpec((K, 1), lambda i: (0, 0)),
        ],
        out_specs=pl.BlockSpec((L, Bb), lambda i: (0, i)),
        out_shape=jax.ShapeDtypeStruct((L, B), jnp.float32),
        scratch_shapes=[
            pltpu.VMEM((L // _LG, 3 * K, _LG), jnp.float32),
        ],
    )(zt, means, logvars, w2)
    return out.T


# speedup vs baseline: 1.4071x; 1.4071x over previous
import functools
import math

import jax
import jax.numpy as jnp
from jax import lax
from jax.experimental import pallas as pl
from jax.experimental.pallas import tpu as pltpu

_HALF_LOG_2PI = 0.5 * math.log(2.0 * math.pi)
_LOG2E = 1.4426950408889634
_LN2 = 0.6931471805599453
_LG = 8
_KC = 32


def _body(zt_ref, mu_ref, lv_ref, w_ref, out_ref, p3_s):
    K, L = mu_ref.shape
    Bb = zt_ref.shape[1]
    NG = L // _LG
    mu = mu_ref[...]
    lv = lv_ref[...]
    wv = w_ref[...]
    wmax = jnp.max(wv)
    logw = wv - wmax - jnp.log(jnp.sum(jnp.exp(wv - wmax)))
    prec = jnp.exp(-lv)
    a_all = _LOG2E * ((-_HALF_LOG_2PI) - 0.5 * lv
                      - 0.5 * prec * mu * mu + logw)
    b_all = _LOG2E * prec * mu
    c_all = (-0.5 * _LOG2E) * prec
    for g in range(NG):
        sl = slice(g * _LG, (g + 1) * _LG)
        p3_s[pl.ds(g, 1)] = jnp.concatenate(
            [a_all[:, sl], b_all[:, sl], c_all[:, sl]], axis=0)[None]

    def lgroup(g, _):
        pc = p3_s[pl.ds(g, 1)][0]
        for j in range(_LG):
            col = pc[:, j:j + 1]
            a = col[0:K]
            b = col[K:2 * K]
            c = col[2 * K:3 * K]
            zrow = zt_ref[pl.ds(g * _LG + j, 1), :]
            z2 = zrow * zrow
            m_run = None
            s_run = None
            for kc in range(K // _KC):
                ks = slice(kc * _KC, (kc + 1) * _KC)
                t2c = a[ks] + b[ks] * zrow + c[ks] * z2
                t3 = t2c.reshape(_KC // 8, 8, Bb)
                mc = jnp.max(t3, axis=0)
                sc = jnp.sum(jnp.exp2(t3 - mc[None]), axis=0)
                if m_run is None:
                    m_run, s_run = mc, sc
                else:
                    m_new = jnp.maximum(m_run, mc)
                    s_run = (s_run * jnp.exp2(m_run - m_new)
                             + sc * jnp.exp2(mc - m_new))
                    m_run = m_new
            m1 = jnp.max(m_run, axis=0, keepdims=True)
            s1 = jnp.sum(s_run * jnp.exp2(m_run - m1),
                         axis=0, keepdims=True)
            out_ref[pl.ds(g * _LG + j, 1), :] = _LN2 * (m1 + jnp.log2(s1))
        return 0

    lax.fori_loop(0, NG, lgroup, 0)


@jax.jit
def kernel(z, means, logvars, w):
    B, L = z.shape
    K = means.shape[0]
    zt = z.T
    w2 = w.reshape(K, 1)
    Bb = 1024
    grid = (B // Bb,)
    out = pl.pallas_call(
        _body,
        grid=grid,
        in_specs=[
            pl.BlockSpec((L, Bb), lambda i: (0, i)),
            pl.BlockSpec((K, L), lambda i: (0, 0)),
            pl.BlockSpec((K, L), lambda i: (0, 0)),
            pl.BlockSpec((K, 1), lambda i: (0, 0)),
        ],
        out_specs=pl.BlockSpec((L, Bb), lambda i: (0, i)),
        out_shape=jax.ShapeDtypeStruct((L, B), jnp.float32),
        scratch_shapes=[
            pltpu.VMEM((L // _LG, 3 * K, _LG), jnp.float32),
        ],
    )(zt, means, logvars, w2)
    return out.T
